# Initial kernel scaffold; baseline (speedup 1.0000x reference)
import jax
import jax.numpy as jnp
from jax.experimental import pallas as pl


def kernel(target_ids, context_ids, negative_ids, input_emb, output_emb):
    raise NotImplementedError("write your pallas kernel here")


# speedup vs baseline: 4.9624x; 4.9624x over previous
import functools

import jax
import jax.numpy as jnp
from jax import lax
from jax.experimental import pallas as pl
from jax.experimental.pallas import tpu as pltpu, tpu_sc as plsc

VOCAB = 1000000
D = 64
B = 16384
C = 20
K = 20

NC = 2
NS = 16
NW = NC * NS
ROWS_W = B // NW
R = 32
NCHUNK = ROWS_W // R
IDX_BLK = 128
NBLK = R * C // IDX_BLK


def _cbow_body(tgt_hbm, ctxidx_hbm, negidx_hbm, inemb_hbm, outemb_hbm,
               out_hbm, idx_ctx_v, idx_neg_v, idx_tgt_v, ctx_rows_v,
               orow_v, out_v, sem):
    wid = lax.axis_index("s") * NC + lax.axis_index("c")
    lane = lax.iota(jnp.int32, 16)

    def chunk_body(ch, carry):
        row0 = wid * ROWS_W + ch * R
        off = row0 * C

        for j in range(NBLK):
            pltpu.sync_copy(ctxidx_hbm.at[pl.ds(off + j * IDX_BLK, IDX_BLK)],
                            idx_ctx_v.at[j])
            pltpu.sync_copy(negidx_hbm.at[pl.ds(off + j * IDX_BLK, IDX_BLK)],
                            idx_neg_v.at[j])
        pltpu.sync_copy(tgt_hbm.at[pl.ds(row0, R)], idx_tgt_v)

        copies = []
        for j in range(NBLK):
            copies.append(pltpu.async_copy(
                inemb_hbm.at[idx_ctx_v.at[j]],
                ctx_rows_v.at[pl.ds(j * IDX_BLK, IDX_BLK)], sem))
            copies.append(pltpu.async_copy(
                outemb_hbm.at[idx_neg_v.at[j]],
                orow_v.at[pl.ds(j * IDX_BLK, IDX_BLK)], sem))
        copies.append(pltpu.async_copy(
            outemb_hbm.at[idx_tgt_v], orow_v.at[pl.ds(R * K, R)], sem))
        for cp in copies:
            cp.wait()

        def row_body(r, rcarry):
            base = r * C
            acc = [ctx_rows_v[base, pl.ds(j * 16, 16)] for j in range(4)]
            for c in range(1, C):
                for j in range(4):
                    acc[j] = acc[j] + ctx_rows_v[base + c, pl.ds(j * 16, 16)]
            inv_c = jnp.float32(1.0 / C)
            ctxv = [acc[j] * inv_c for j in range(4)]

            def dot(row):
                v = ctxv[0] * orow_v[row, pl.ds(0, 16)]
                for j in range(1, 4):
                    v = v + ctxv[j] * orow_v[row, pl.ds(j * 16, 16)]
                return jnp.sum(v)

            acc1 = jnp.zeros((16,), jnp.float32)
            acc2 = jnp.where(lane == 4, dot(R * K + r),
                             jnp.zeros((16,), jnp.float32))
            for k in range(K):
                s = dot(base + k)
                if k < 16:
                    acc1 = jnp.where(lane == k, s, acc1)
                else:
                    acc2 = jnp.where(lane == k - 16, s, acc2)
            out_v[r, pl.ds(0, 16)] = acc1
            out_v[r, pl.ds(16, 16)] = acc2
            return rcarry

        lax.fori_loop(0, R, row_body, 0)
        pltpu.sync_copy(out_v, out_hbm.at[pl.ds(row0, R)])
        return carry

    lax.fori_loop(0, NCHUNK, chunk_body, 0)


@functools.partial(jax.jit, static_argnums=())
def _cbow_sc(target_ids, ctx_idx, neg_idx, input_emb, output_emb):
    mesh = plsc.VectorSubcoreMesh(core_axis_name="c", subcore_axis_name="s")
    k = pl.kernel(
        _cbow_body,
        mesh=mesh,
        compiler_params=pltpu.CompilerParams(
            needs_layout_passes=False, use_tc_tiling_on_sc=False),
        out_type=jax.ShapeDtypeStruct((B, 32), jnp.float32),
        scratch_types=[
            pltpu.VMEM((NBLK, IDX_BLK), jnp.int32),
            pltpu.VMEM((NBLK, IDX_BLK), jnp.int32),
            pltpu.VMEM((R,), jnp.int32),
            pltpu.VMEM((R * C, D), jnp.float32),
            pltpu.VMEM((R * K + R, D), jnp.float32),
            pltpu.VMEM((R, 32), jnp.float32),
            pltpu.SemaphoreType.DMA,
        ],
    )
    return k(target_ids, ctx_idx, neg_idx, input_emb, output_emb)


def kernel(target_ids, context_ids, negative_ids, input_emb, output_emb):
    ctx_idx = context_ids.astype(jnp.int32).reshape(B * C)
    neg_idx = negative_ids.astype(jnp.int32).reshape(B * K)
    out = _cbow_sc(target_ids.astype(jnp.int32), ctx_idx, neg_idx,
                   input_emb, output_emb)
    return (out[:, K], out[:, :K])


# speedup vs baseline: 5.1264x; 1.0331x over previous
import functools

import jax
import jax.numpy as jnp
from jax import lax
from jax.experimental import pallas as pl
from jax.experimental.pallas import tpu as pltpu, tpu_sc as plsc

VOCAB = 1000000
D = 64
DP = 128
B = 16384
C = 20
K = 20

NC = 2
NS = 16
NW = NC * NS
ROWS_W = B // NW
R = 16
NCHUNK = ROWS_W // R
IDX_BLK = 80
NBLK = R * C // IDX_BLK


def _cbow_body(tgt_hbm, ctxidx_hbm, negidx_hbm, inemb_hbm, outemb_hbm,
               out_hbm, idx_ctx_v, idx_neg_v, idx_tgt_v, ctx_rows_v,
               orow_v, out_v, sem):
    wid = lax.axis_index("s") * NC + lax.axis_index("c")
    lane = lax.iota(jnp.int32, 16)

    def chunk_body(ch, carry):
        row0 = wid * ROWS_W + ch * R
        off = row0 * C

        pltpu.sync_copy(ctxidx_hbm.at[pl.ds(off, R * C)], idx_ctx_v)
        pltpu.sync_copy(negidx_hbm.at[pl.ds(off, R * K)], idx_neg_v)
        pltpu.sync_copy(tgt_hbm.at[pl.ds(row0, R)], idx_tgt_v)

        copies = []
        for j in range(NBLK):
            copies.append(pltpu.async_copy(
                inemb_hbm.at[idx_ctx_v.at[pl.ds(j * IDX_BLK, IDX_BLK)]],
                ctx_rows_v.at[pl.ds(j * IDX_BLK, IDX_BLK)], sem))
            copies.append(pltpu.async_copy(
                outemb_hbm.at[idx_neg_v.at[pl.ds(j * IDX_BLK, IDX_BLK)]],
                orow_v.at[pl.ds(j * IDX_BLK, IDX_BLK)], sem))
        copies.append(pltpu.async_copy(
            outemb_hbm.at[idx_tgt_v], orow_v.at[pl.ds(R * K, R)], sem))
        for cp in copies:
            cp.wait()

        def row_body(r, rcarry):
            base = r * C
            acc = [ctx_rows_v[base, pl.ds(j * 16, 16)] for j in range(4)]
            for c in range(1, C):
                for j in range(4):
                    acc[j] = acc[j] + ctx_rows_v[base + c, pl.ds(j * 16, 16)]
            inv_c = jnp.float32(1.0 / C)
            ctxv = [acc[j] * inv_c for j in range(4)]

            def dot(row):
                v = ctxv[0] * orow_v[row, pl.ds(0, 16)]
                for j in range(1, 4):
                    v = v + ctxv[j] * orow_v[row, pl.ds(j * 16, 16)]
                return jnp.sum(v)

            acc1 = jnp.zeros((16,), jnp.float32)
            acc2 = jnp.where(lane == 4, dot(R * K + r),
                             jnp.zeros((16,), jnp.float32))
            for k in range(K):
                s = dot(base + k)
                if k < 16:
                    acc1 = jnp.where(lane == k, s, acc1)
                else:
                    acc2 = jnp.where(lane == k - 16, s, acc2)
            out_v[r, pl.ds(0, 16)] = acc1
            out_v[r, pl.ds(16, 16)] = acc2
            return rcarry

        lax.fori_loop(0, R, row_body, 0)
        pltpu.sync_copy(out_v, out_hbm.at[pl.ds(row0, R)])
        return carry

    lax.fori_loop(0, NCHUNK, chunk_body, 0)


@functools.partial(jax.jit, static_argnums=())
def _cbow_sc(target_ids, ctx_idx, neg_idx, inemb_p, outemb_p):
    mesh = plsc.VectorSubcoreMesh(core_axis_name="c", subcore_axis_name="s")
    k = pl.kernel(
        _cbow_body,
        mesh=mesh,
        compiler_params=pltpu.CompilerParams(needs_layout_passes=False),
        out_type=jax.ShapeDtypeStruct((B, 32), jnp.float32),
        scratch_types=[
            pltpu.VMEM((R * C,), jnp.int32),
            pltpu.VMEM((R * K,), jnp.int32),
            pltpu.VMEM((R,), jnp.int32),
            pltpu.VMEM((R * C, DP), jnp.float32),
            pltpu.VMEM((R * K + R, DP), jnp.float32),
            pltpu.VMEM((R, 32), jnp.float32),
            pltpu.SemaphoreType.DMA,
        ],
    )
    return k(target_ids, ctx_idx, neg_idx, inemb_p, outemb_p)


def kernel(target_ids, context_ids, negative_ids, input_emb, output_emb):
    ctx_idx = context_ids.astype(jnp.int32).reshape(B * C)
    neg_idx = negative_ids.astype(jnp.int32).reshape(B * K)
    inemb_p = jnp.pad(input_emb, ((0, 0), (0, DP - D)))
    outemb_p = jnp.pad(output_emb, ((0, 0), (0, DP - D)))
    out = _cbow_sc(target_ids.astype(jnp.int32), ctx_idx, neg_idx,
                   inemb_p, outemb_p)
    return (out[:, K], out[:, :K])


# speedup vs baseline: 5.6801x; 1.1080x over previous
import functools

import jax
import jax.numpy as jnp
from jax import lax
from jax.experimental import pallas as pl
from jax.experimental.pallas import tpu as pltpu, tpu_sc as plsc

VOCAB = 1000000
D = 64
DP = 128
B = 16384
C = 20
K = 20

NC = 2
NS = 16
NW = NC * NS
ROWS_W = B // NW
R = 16
NCHUNK = ROWS_W // R
IDX_BLK = 80
NBLK = R * C // IDX_BLK


def _cbow_body(tgt_hbm, ctxidx_hbm, negidx_hbm, comb_hbm,
               out_hbm, idx_ctx_v, idx_neg_v, idx_tgt_v, ctx_rows_v,
               orow_v, out_v, sem):
    wid = lax.axis_index("s") * NC + lax.axis_index("c")
    lane = lax.iota(jnp.int32, 16)

    def chunk_body(ch, carry):
        row0 = wid * ROWS_W + ch * R
        off = row0 * C

        pltpu.sync_copy(ctxidx_hbm.at[pl.ds(off, R * C)], idx_ctx_v)
        pltpu.sync_copy(negidx_hbm.at[pl.ds(off, R * K)], idx_neg_v)
        pltpu.sync_copy(tgt_hbm.at[pl.ds(row0, R)], idx_tgt_v)

        copies = []
        for j in range(NBLK):
            copies.append(pltpu.async_copy(
                comb_hbm.at[idx_ctx_v.at[pl.ds(j * IDX_BLK, IDX_BLK)]],
                ctx_rows_v.at[pl.ds(j * IDX_BLK, IDX_BLK)], sem))
            copies.append(pltpu.async_copy(
                comb_hbm.at[idx_neg_v.at[pl.ds(j * IDX_BLK, IDX_BLK)]],
                orow_v.at[pl.ds(j * IDX_BLK, IDX_BLK)], sem))
        copies.append(pltpu.async_copy(
            comb_hbm.at[idx_tgt_v], orow_v.at[pl.ds(R * K, R)], sem))
        for cp in copies:
            cp.wait()

        def row_body(r, rcarry):
            base = r * C
            acc = [ctx_rows_v[base, pl.ds(j * 16, 16)] for j in range(4)]
            for c in range(1, C):
                for j in range(4):
                    acc[j] = acc[j] + ctx_rows_v[base + c, pl.ds(j * 16, 16)]
            inv_c = jnp.float32(1.0 / C)
            ctxv = [acc[j] * inv_c for j in range(4)]

            def dot(row):
                v = ctxv[0] * orow_v[row, pl.ds(D, 16)]
                for j in range(1, 4):
                    v = v + ctxv[j] * orow_v[row, pl.ds(D + j * 16, 16)]
                return jnp.sum(v)

            acc1 = jnp.zeros((16,), jnp.float32)
            acc2 = jnp.where(lane == 4, dot(R * K + r),
                             jnp.zeros((16,), jnp.float32))
            for k in range(K):
                s = dot(base + k)
                if k < 16:
                    acc1 = jnp.where(lane == k, s, acc1)
                else:
                    acc2 = jnp.where(lane == k - 16, s, acc2)
            out_v[r, pl.ds(0, 16)] = acc1
            out_v[r, pl.ds(16, 16)] = acc2
            return rcarry

        lax.fori_loop(0, R, row_body, 0)
        pltpu.sync_copy(out_v, out_hbm.at[pl.ds(row0, R)])
        return carry

    lax.fori_loop(0, NCHUNK, chunk_body, 0)


@functools.partial(jax.jit, static_argnums=())
def _cbow_sc(target_ids, ctx_idx, neg_idx, comb):
    mesh = plsc.VectorSubcoreMesh(core_axis_name="c", subcore_axis_name="s")
    k = pl.kernel(
        _cbow_body,
        mesh=mesh,
        compiler_params=pltpu.CompilerParams(needs_layout_passes=False),
        out_type=jax.ShapeDtypeStruct((B, 32), jnp.float32),
        scratch_types=[
            pltpu.VMEM((R * C,), jnp.int32),
            pltpu.VMEM((R * K,), jnp.int32),
            pltpu.VMEM((R,), jnp.int32),
            pltpu.VMEM((R * C, DP), jnp.float32),
            pltpu.VMEM((R * K + R, DP), jnp.float32),
            pltpu.VMEM((R, 32), jnp.float32),
            pltpu.SemaphoreType.DMA,
        ],
    )
    return k(target_ids, ctx_idx, neg_idx, comb)


def kernel(target_ids, context_ids, negative_ids, input_emb, output_emb):
    ctx_idx = context_ids.astype(jnp.int32).reshape(B * C)
    neg_idx = negative_ids.astype(jnp.int32).reshape(B * K)
    comb = jnp.concatenate([input_emb, output_emb], axis=1)
    out = _cbow_sc(target_ids.astype(jnp.int32), ctx_idx, neg_idx, comb)
    return (out[:, K], out[:, :K])
